# baseline (speedup 1.0000x reference)
import functools

import jax
import jax.numpy as jnp
from jax import lax
from jax.experimental import pallas as pl
from jax.experimental.pallas import tpu as pltpu
from jax.experimental.pallas import tpu_sc as plsc


def _pos_encoding(seq_len, d_model):
    pos = jnp.arange(seq_len, dtype=jnp.float32)[:, None]
    two_i = jnp.arange(0, d_model, 2, dtype=jnp.float32)
    div = jnp.power(10000.0, two_i / d_model)
    enc = jnp.zeros((seq_len, d_model), dtype=jnp.float32)
    enc = enc.at[:, 0::2].set(jnp.sin(pos / div))
    enc = enc.at[:, 1::2].set(jnp.cos(pos / div))
    return enc


@functools.lru_cache(maxsize=None)
def _make_sc_kernel(B, S, D):
    info = plsc.get_sparse_core_info()
    NC, NS, L = info.num_cores, info.num_subcores, info.num_lanes
    NW = NC * NS
    N = B * S
    assert N % NW == 0
    per_w = N // NW
    CHUNK = 64
    assert per_w % CHUNK == 0
    nchunks = per_w // CHUNK
    mesh = plsc.VectorSubcoreMesh(core_axis_name="c", subcore_axis_name="s")

    @functools.partial(
        pl.kernel,
        mesh=mesh,
        out_type=jax.ShapeDtypeStruct((N, D), jnp.float32),
        scratch_types=[
            pltpu.VMEM((per_w,), jnp.int32),
            pltpu.VMEM((S, D), jnp.float32),
            pltpu.VMEM((CHUNK, D), jnp.float32),
            pltpu.SemaphoreType.DMA,
        ],
    )
    def emb_kernel(idx_hbm, table_hbm, pe_hbm, out_hbm, idx_v, pe_v, buf, sem):
        wid = lax.axis_index("s") * NC + lax.axis_index("c")
        base = wid * per_w
        pltpu.sync_copy(idx_hbm.at[pl.ds(base, per_w)], idx_v)
        pltpu.sync_copy(pe_hbm, pe_v)

        def chunk_body(c, carry):
            row0 = base + c * CHUNK
            pltpu.async_copy(
                table_hbm.at[idx_v.at[pl.ds(c * CHUNK, CHUNK)]], buf, sem
            ).wait()

            def row_body(r, carry2):
                s_pos = lax.rem(row0 + r, S)
                for j in range(D // L):
                    sl = pl.ds(j * L, L)
                    buf[r, sl] = buf[r, sl] + pe_v[s_pos, sl]
                return carry2

            lax.fori_loop(0, CHUNK, row_body, 0)
            pltpu.sync_copy(buf, out_hbm.at[pl.ds(row0, CHUNK)])
            return carry

        lax.fori_loop(0, nchunks, chunk_body, 0)

    return emb_kernel


def kernel(x, token_table):
    B, S = x.shape
    D = token_table.shape[1]
    pe = _pos_encoding(S, D)
    out = _make_sc_kernel(B, S, D)(x.reshape(-1), token_table, pe)
    return out.reshape(B, S, D)


# speedup vs baseline: 1.1848x; 1.1848x over previous
import functools

import jax
import jax.numpy as jnp
from jax import lax
from jax.experimental import pallas as pl
from jax.experimental.pallas import tpu as pltpu
from jax.experimental.pallas import tpu_sc as plsc


def _pos_encoding(seq_len, d_model):
    pos = jnp.arange(seq_len, dtype=jnp.float32)[:, None]
    two_i = jnp.arange(0, d_model, 2, dtype=jnp.float32)
    div = jnp.power(10000.0, two_i / d_model)
    enc = jnp.zeros((seq_len, d_model), dtype=jnp.float32)
    enc = enc.at[:, 0::2].set(jnp.sin(pos / div))
    enc = enc.at[:, 1::2].set(jnp.cos(pos / div))
    return enc


@functools.lru_cache(maxsize=None)
def _make_sc_kernel(B, S, D):
    info = plsc.get_sparse_core_info()
    NC, NS, L = info.num_cores, info.num_subcores, info.num_lanes
    NW = NC * NS
    N = B * S
    assert N % NW == 0
    per_w = N // NW
    CHUNK = 40
    assert per_w % CHUNK == 0
    nchunks = per_w // CHUNK
    assert nchunks % 2 == 0 and nchunks >= 6
    mesh = plsc.VectorSubcoreMesh(core_axis_name="c", subcore_axis_name="s")

    @functools.partial(
        pl.kernel,
        mesh=mesh,
        out_type=jax.ShapeDtypeStruct((N, D), jnp.float32),
        scratch_types=[
            pltpu.VMEM((per_w,), jnp.int32),
            pltpu.VMEM((S, D), jnp.float32),
            pltpu.VMEM((2, CHUNK, D), jnp.float32),
            pltpu.VMEM((2, CHUNK, D), jnp.float32),
            pltpu.SemaphoreType.DMA,
            pltpu.SemaphoreType.DMA,
            pltpu.SemaphoreType.DMA,
            pltpu.SemaphoreType.DMA,
        ],
    )
    def emb_kernel(idx_hbm, table_hbm, pe_hbm, out_hbm,
                   idx_v, pe_v, gbuf, obuf, g0, g1, o0, o1):
        wid = lax.axis_index("s") * NC + lax.axis_index("c")
        base = wid * per_w
        gsem = (g0, g1)
        osem = (o0, o1)

        pltpu.sync_copy(idx_hbm.at[pl.ds(base, per_w)], idx_v)

        def start_gather(c, b):
            pltpu.async_copy(
                table_hbm.at[idx_v.at[pl.ds(c * CHUNK, CHUNK)]],
                gbuf.at[b], gsem[b])

        def start_out(c, b):
            pltpu.async_copy(
                obuf.at[b], out_hbm.at[pl.ds(base + c * CHUNK, CHUNK)], osem[b])

        def wait_gather(b):
            pltpu.make_async_copy(table_hbm.at[idx_v.at[pl.ds(0, CHUNK)]],
                                  gbuf.at[b], gsem[b]).wait()

        def wait_out(b):
            pltpu.make_async_copy(obuf.at[b],
                                  out_hbm.at[pl.ds(base, CHUNK)], osem[b]).wait()

        def add_pe(c, b):
            row0 = base + c * CHUNK

            def row_body(r, carry):
                s_pos = lax.rem(row0 + r, S)
                for j in range(D // L):
                    sl = pl.ds(j * L, L)
                    obuf[b, r, sl] = gbuf[b, r, sl] + pe_v[s_pos, sl]
                return carry

            lax.fori_loop(0, CHUNK, row_body, 0)

        start_gather(0, 0)
        start_gather(1, 1)
        pltpu.sync_copy(pe_hbm, pe_v)

        for b in range(2):
            wait_gather(b)
            add_pe(b, b)
            start_gather(b + 2, b)
            start_out(b, b)

        def outer_body(o, carry):
            for b in range(2):
                c = 2 * o + b
                wait_gather(b)
                wait_out(b)
                add_pe(c, b)
                start_gather(c + 2, b)
                start_out(c, b)
            return carry

        lax.fori_loop(1, nchunks // 2 - 1, outer_body, 0)

        for b in range(2):
            c = nchunks - 2 + b
            wait_gather(b)
            wait_out(b)
            add_pe(c, b)
            start_out(c, b)

        wait_out(0)
        wait_out(1)

    return emb_kernel


def kernel(x, token_table):
    B, S = x.shape
    D = token_table.shape[1]
    pe = _pos_encoding(S, D)
    out = _make_sc_kernel(B, S, D)(x.reshape(-1), token_table, pe)
    return out.reshape(B, S, D)


# speedup vs baseline: 1.9463x; 1.6427x over previous
import functools

import jax
import jax.numpy as jnp
from jax import lax
from jax.experimental import pallas as pl
from jax.experimental.pallas import tpu as pltpu
from jax.experimental.pallas import tpu_sc as plsc


def _pos_encoding(seq_len, d_model):
    pos = jnp.arange(seq_len, dtype=jnp.float32)[:, None]
    two_i = jnp.arange(0, d_model, 2, dtype=jnp.float32)
    div = jnp.power(10000.0, two_i / d_model)
    enc = jnp.zeros((seq_len, d_model), dtype=jnp.float32)
    enc = enc.at[:, 0::2].set(jnp.sin(pos / div))
    enc = enc.at[:, 1::2].set(jnp.cos(pos / div))
    return enc


@functools.lru_cache(maxsize=None)
def _make_sc_kernel(B, S, D):
    info = plsc.get_sparse_core_info()
    NC, NS, L = info.num_cores, info.num_subcores, info.num_lanes
    NW = NC * NS
    N = B * S
    assert N % NW == 0
    per_w = N // NW
    CHUNK = 40
    assert per_w % CHUNK == 0
    nchunks = per_w // CHUNK
    assert nchunks % 2 == 0 and nchunks >= 6
    mesh = plsc.VectorSubcoreMesh(core_axis_name="c", subcore_axis_name="s")

    @functools.partial(
        pl.kernel,
        mesh=mesh,
        out_type=jax.ShapeDtypeStruct((N, D), jnp.float32),
        scratch_types=[
            pltpu.VMEM((per_w,), jnp.int32),
            pltpu.VMEM((S, D), jnp.float32),
            pltpu.VMEM((2, CHUNK, D), jnp.float32),
            pltpu.VMEM((2, CHUNK, D), jnp.float32),
            pltpu.SemaphoreType.DMA,
            pltpu.SemaphoreType.DMA,
            pltpu.SemaphoreType.DMA,
            pltpu.SemaphoreType.DMA,
        ],
    )
    def emb_kernel(idx_hbm, table_hbm, pe_hbm, out_hbm,
                   idx_v, pe_v, gbuf, obuf, g0, g1, o0, o1):
        wid = lax.axis_index("s") * NC + lax.axis_index("c")
        base = wid * per_w
        gsem = (g0, g1)
        osem = (o0, o1)

        pltpu.sync_copy(idx_hbm.at[pl.ds(base, per_w)], idx_v)

        def start_gather(c, b):
            pltpu.async_copy(
                table_hbm.at[idx_v.at[pl.ds(c * CHUNK, CHUNK)]],
                gbuf.at[b], gsem[b])

        def start_out(c, b):
            pltpu.async_copy(
                obuf.at[b], out_hbm.at[pl.ds(base + c * CHUNK, CHUNK)], osem[b])

        def wait_gather(b):
            pltpu.make_async_copy(table_hbm.at[idx_v.at[pl.ds(0, CHUNK)]],
                                  gbuf.at[b], gsem[b]).wait()

        def wait_out(b):
            pltpu.make_async_copy(obuf.at[b],
                                  out_hbm.at[pl.ds(base, CHUNK)], osem[b]).wait()

        def add_pe(c, b):
            row0 = base + c * CHUNK

            def row_body(r, carry):
                s_pos = (row0 + r) // B
                for j in range(D // L):
                    sl = pl.ds(j * L, L)
                    obuf[b, r, sl] = gbuf[b, r, sl] + pe_v[s_pos, sl]
                return carry

            lax.fori_loop(0, CHUNK, row_body, 0)

        start_gather(0, 0)
        start_gather(1, 1)
        pltpu.sync_copy(pe_hbm, pe_v)

        for b in range(2):
            wait_gather(b)
            add_pe(b, b)
            start_gather(b + 2, b)
            start_out(b, b)

        def outer_body(o, carry):
            for b in range(2):
                c = 2 * o + b
                wait_gather(b)
                wait_out(b)
                add_pe(c, b)
                start_gather(c + 2, b)
                start_out(c, b)
            return carry

        lax.fori_loop(1, nchunks // 2 - 1, outer_body, 0)

        for b in range(2):
            c = nchunks - 2 + b
            wait_gather(b)
            wait_out(b)
            add_pe(c, b)
            start_out(c, b)

        wait_out(0)
        wait_out(1)

    return emb_kernel


def kernel(x, token_table):
    B, S = x.shape
    D = token_table.shape[1]
    pe = _pos_encoding(S, D)
    idx = x.T.reshape(-1)
    out = _make_sc_kernel(B, S, D)(idx, token_table, pe)
    return out.reshape(S, B, D).transpose(1, 0, 2)


# speedup vs baseline: 5.0540x; 2.5967x over previous
import functools

import jax
import jax.numpy as jnp
from jax import lax
from jax.experimental import pallas as pl
from jax.experimental.pallas import tpu as pltpu
from jax.experimental.pallas import tpu_sc as plsc


def _pos_encoding(seq_len, d_model):
    pos = jnp.arange(seq_len, dtype=jnp.float32)[:, None]
    two_i = jnp.arange(0, d_model, 2, dtype=jnp.float32)
    div = jnp.power(10000.0, two_i / d_model)
    enc = jnp.zeros((seq_len, d_model), dtype=jnp.float32)
    enc = enc.at[:, 0::2].set(jnp.sin(pos / div))
    enc = enc.at[:, 1::2].set(jnp.cos(pos / div))
    return enc


@functools.lru_cache(maxsize=None)
def _make_sc_kernel(B, S, D):
    info = plsc.get_sparse_core_info()
    NC, NS, L = info.num_cores, info.num_subcores, info.num_lanes
    NW = NC * NS
    N = B * S
    assert N % NW == 0
    per_w = N // NW
    CHUNK = 40
    assert per_w % CHUNK == 0
    nchunks = per_w // CHUNK
    assert nchunks % 2 == 0 and nchunks >= 6
    mesh = plsc.VectorSubcoreMesh(core_axis_name="c", subcore_axis_name="s")

    @functools.partial(
        pl.kernel,
        mesh=mesh,
        out_type=jax.ShapeDtypeStruct((N, D), jnp.float32),
        scratch_types=[
            pltpu.VMEM((per_w,), jnp.int32),
            pltpu.VMEM((S, D), jnp.float32),
            pltpu.VMEM((2, CHUNK, D), jnp.float32),
            pltpu.VMEM((2, CHUNK, D), jnp.float32),
            pltpu.SemaphoreType.DMA,
            pltpu.SemaphoreType.DMA,
            pltpu.SemaphoreType.DMA,
            pltpu.SemaphoreType.DMA,
        ],
    )
    def emb_kernel(idx_hbm, table_hbm, pe_hbm, out_hbm,
                   idx_v, pe_v, gbuf, obuf, g0, g1, o0, o1):
        wid = lax.axis_index("s") * NC + lax.axis_index("c")
        base = wid * per_w
        gsem = (g0, g1)
        osem = (o0, o1)

        pltpu.sync_copy(idx_hbm.at[pl.ds(base, per_w)], idx_v)

        def start_gather(c, b):
            pltpu.async_copy(
                table_hbm.at[idx_v.at[pl.ds(c * CHUNK, CHUNK)]],
                gbuf.at[b], gsem[b])

        def start_out(c, b):
            pltpu.async_copy(
                obuf.at[b], out_hbm.at[pl.ds(base + c * CHUNK, CHUNK)], osem[b])

        def wait_gather(b):
            pltpu.make_async_copy(table_hbm.at[idx_v.at[pl.ds(0, CHUNK)]],
                                  gbuf.at[b], gsem[b]).wait()

        def wait_out(b):
            pltpu.make_async_copy(obuf.at[b],
                                  out_hbm.at[pl.ds(base, CHUNK)], osem[b]).wait()

        def add_pe(c, b):
            row0 = base + c * CHUNK
            s0 = row0 // B
            m = jnp.minimum((s0 + 1) * B - row0, CHUNK)
            s1 = jnp.minimum(s0 + 1, S - 1)

            def add_run(rlo, rhi, s_fixed):
                pes = [pe_v[s_fixed, pl.ds(j * L, L)] for j in range(D // L)]

                def row_body(r, carry):
                    for j in range(D // L):
                        sl = pl.ds(j * L, L)
                        obuf[b, r, sl] = gbuf[b, r, sl] + pes[j]
                    return carry

                lax.fori_loop(rlo, rhi, row_body, 0)

            add_run(0, m, s0)
            add_run(m, CHUNK, s1)

        start_gather(0, 0)
        start_gather(1, 1)
        pltpu.sync_copy(pe_hbm, pe_v)

        for b in range(2):
            wait_gather(b)
            add_pe(b, b)
            start_gather(b + 2, b)
            start_out(b, b)

        def outer_body(o, carry):
            for b in range(2):
                c = 2 * o + b
                wait_gather(b)
                wait_out(b)
                add_pe(c, b)
                start_gather(c + 2, b)
                start_out(c, b)
            return carry

        lax.fori_loop(1, nchunks // 2 - 1, outer_body, 0)

        for b in range(2):
            c = nchunks - 2 + b
            wait_gather(b)
            wait_out(b)
            add_pe(c, b)
            start_out(c, b)

        wait_out(0)
        wait_out(1)

    return emb_kernel


def kernel(x, token_table):
    B, S = x.shape
    D = token_table.shape[1]
    pe = _pos_encoding(S, D)
    idx = x.T.reshape(-1)
    out = _make_sc_kernel(B, S, D)(idx, token_table, pe)
    return out.reshape(S, B, D).transpose(1, 0, 2)


# speedup vs baseline: 5.1690x; 1.0228x over previous
import functools

import jax
import jax.numpy as jnp
import numpy as np
from jax import lax
from jax.experimental import pallas as pl
from jax.experimental.pallas import tpu as pltpu
from jax.experimental.pallas import tpu_sc as plsc


def _pos_encoding(seq_len, d_model):
    pos = np.arange(seq_len, dtype=np.float32)[:, None]
    two_i = np.arange(0, d_model, 2, dtype=np.float32)
    div = np.power(np.float32(10000.0), two_i / np.float32(d_model))
    enc = np.zeros((seq_len, d_model), dtype=np.float32)
    enc[:, 0::2] = np.sin(pos / div)
    enc[:, 1::2] = np.cos(pos / div)
    return jnp.asarray(enc)


@functools.lru_cache(maxsize=None)
def _make_sc_kernel(B, S, D):
    info = plsc.get_sparse_core_info()
    NC, NS, L = info.num_cores, info.num_subcores, info.num_lanes
    NW = NC * NS
    N = B * S
    assert N % NW == 0
    per_w = N // NW
    CHUNK = 40
    assert per_w % CHUNK == 0
    nchunks = per_w // CHUNK
    assert nchunks % 2 == 0 and nchunks >= 6
    mesh = plsc.VectorSubcoreMesh(core_axis_name="c", subcore_axis_name="s")

    @functools.partial(
        pl.kernel,
        mesh=mesh,
        out_type=jax.ShapeDtypeStruct((N, D), jnp.float32),
        scratch_types=[
            pltpu.VMEM((per_w,), jnp.int32),
            pltpu.VMEM((S, D), jnp.float32),
            pltpu.VMEM((2, CHUNK, D), jnp.float32),
            pltpu.VMEM((2, CHUNK, D), jnp.float32),
            pltpu.SemaphoreType.DMA,
            pltpu.SemaphoreType.DMA,
            pltpu.SemaphoreType.DMA,
            pltpu.SemaphoreType.DMA,
        ],
    )
    def emb_kernel(idx_hbm, table_hbm, pe_hbm, out_hbm,
                   idx_v, pe_v, gbuf, obuf, g0, g1, o0, o1):
        wid = lax.axis_index("s") * NC + lax.axis_index("c")
        base = wid * per_w
        gsem = (g0, g1)
        osem = (o0, o1)

        pltpu.sync_copy(idx_hbm.at[pl.ds(base, per_w)], idx_v)

        def start_gather(c, b):
            pltpu.async_copy(
                table_hbm.at[idx_v.at[pl.ds(c * CHUNK, CHUNK)]],
                gbuf.at[b], gsem[b])

        def start_out(c, b):
            pltpu.async_copy(
                obuf.at[b], out_hbm.at[pl.ds(base + c * CHUNK, CHUNK)], osem[b])

        def wait_gather(b):
            pltpu.make_async_copy(table_hbm.at[idx_v.at[pl.ds(0, CHUNK)]],
                                  gbuf.at[b], gsem[b]).wait()

        def wait_out(b):
            pltpu.make_async_copy(obuf.at[b],
                                  out_hbm.at[pl.ds(base, CHUNK)], osem[b]).wait()

        def add_pe(c, b):
            row0 = base + c * CHUNK
            s0 = row0 // B
            m = jnp.minimum((s0 + 1) * B - row0, CHUNK)
            s1 = jnp.minimum(s0 + 1, S - 1)

            def add_run(rlo, rhi, s_fixed):
                pes = [pe_v[s_fixed, pl.ds(j * L, L)] for j in range(D // L)]

                def row_body(r, carry):
                    for j in range(D // L):
                        sl = pl.ds(j * L, L)
                        obuf[b, r, sl] = gbuf[b, r, sl] + pes[j]
                    return carry

                lax.fori_loop(rlo, rhi, row_body, 0)

            add_run(0, m, s0)
            add_run(m, CHUNK, s1)

        start_gather(0, 0)
        start_gather(1, 1)
        pltpu.sync_copy(pe_hbm, pe_v)

        for b in range(2):
            wait_gather(b)
            add_pe(b, b)
            start_gather(b + 2, b)
            start_out(b, b)

        def outer_body(o, carry):
            for b in range(2):
                c = 2 * o + b
                wait_gather(b)
                wait_out(b)
                add_pe(c, b)
                start_gather(c + 2, b)
                start_out(c, b)
            return carry

        lax.fori_loop(1, nchunks // 2 - 1, outer_body, 0)

        for b in range(2):
            c = nchunks - 2 + b
            wait_gather(b)
            wait_out(b)
            add_pe(c, b)
            start_out(c, b)

        wait_out(0)
        wait_out(1)

    return emb_kernel


def kernel(x, token_table):
    B, S = x.shape
    D = token_table.shape[1]
    pe = _pos_encoding(S, D)
    idx = x.T.reshape(-1)
    out = _make_sc_kernel(B, S, D)(idx, token_table, pe)
    return out.reshape(S, B, D).transpose(1, 0, 2)
